# Initial kernel scaffold; baseline (speedup 1.0000x reference)
import jax
import jax.numpy as jnp
from jax.experimental import pallas as pl


def kernel(X, C, W_n0, b_n0, W_e0, b_e0, W_e1, b_e1):
    raise NotImplementedError("write your pallas kernel here")


# speedup vs baseline: 14.5338x; 14.5338x over previous
import functools

import jax
import jax.numpy as jnp
from jax.experimental import pallas as pl

BIGV = 1e6
KNN = 32
NRBF = 16


def _rank_kernel(x_ref, c_ref, rank_ref, mi_ref, *, rb, n):
    i = pl.program_id(1)
    key = jnp.where(c_ref[0] > 0, x_ref[0], BIGV)
    ci = c_ref[0, 0:1, pl.ds(i * rb, rb)]
    xi = x_ref[0, 0:1, pl.ds(i * rb, rb)]
    ki = jnp.transpose(jnp.where(ci > 0, xi, BIGV))
    jg = jax.lax.broadcasted_iota(jnp.int32, (1, n), 1)
    ig = jax.lax.broadcasted_iota(jnp.int32, (rb, 1), 0) + i * rb
    less = (key < ki) | ((key == ki) & (jg < ig))
    rank = jnp.sum(less.astype(jnp.int32), axis=1, keepdims=True)
    rank_ref[0] = jnp.transpose(rank)
    mi_ref[0] = (ci > 0).astype(jnp.float32)


def _sortapply_kernel(rank_ref, x_ref, c_ref, sv_ref, si_ref, *, pb, n):
    p = pl.program_id(1)
    key = jnp.where(c_ref[0] > 0, x_ref[0], BIGV)
    rank = rank_ref[0]
    pg = jax.lax.broadcasted_iota(jnp.int32, (pb, 1), 0) + p * pb
    oh = rank == pg
    sv = jnp.sum(jnp.where(oh, key, 0.0), axis=1, keepdims=True)
    jg = jax.lax.broadcasted_iota(jnp.int32, (1, n), 1)
    si = jnp.sum(jnp.where(oh, jg, 0), axis=1, keepdims=True)
    sv_ref[0] = jnp.transpose(sv)
    si_ref[0] = jnp.transpose(si)


def _knn_kernel(svp_ref, sip_ref, et_ref, *, pb, k):
    p0 = pl.program_id(1) * pb
    win_v = svp_ref[0, 0:1, pl.ds(p0, pb + 2 * k)]
    win_i = sip_ref[0, 0:1, pl.ds(p0, pb + 2 * k)]
    rows_v = [win_v[:, w:w + pb] for w in range(2 * k + 1)]
    rows_i = [win_i[:, w:w + pb] for w in range(2 * k + 1)]
    pvals = rows_v[k]
    cv = jnp.concatenate(rows_v[:k] + rows_v[k + 1:], axis=0)
    cidx = jnp.concatenate(rows_i[:k] + rows_i[k + 1:], axis=0)
    dist = jnp.abs(cv - pvals)
    vals, idxs = [], []
    for _ in range(k):
        gmin = jnp.min(dist, axis=0, keepdims=True)
        tie = dist == gmin
        cho = jnp.min(jnp.where(tie, cidx, 2 ** 30), axis=0, keepdims=True)
        vals.append(gmin)
        idxs.append(cho)
        dist = jnp.where(tie & (cidx == cho), 3e6, dist)
    et_ref[0, 0:k, :] = jnp.concatenate(vals, axis=0)
    et_ref[0, k:2 * k, :] = jnp.concatenate(idxs, axis=0).astype(jnp.float32)


def _gatherback_kernel(et_ref, rank_ref, eo_ref, *, nb, n, pc, k):
    acc = jnp.zeros((2 * k, nb), jnp.float32)
    rank = rank_ref[0]
    for cp in range(n // pc):
        pg = jax.lax.broadcasted_iota(jnp.int32, (pc, 1), 0) + cp * pc
        oh = (rank == pg).astype(jnp.float32)
        e = et_ref[0, :, cp * pc:(cp + 1) * pc]
        acc = acc + jnp.dot(e, oh, preferred_element_type=jnp.float32,
                            precision=jax.lax.Precision.HIGHEST)
    eo_ref[0] = acc


def _feat_kernel(eo_ref, x_ref, mi_ref, wn_ref, bn_ref, we0_ref, be0_ref,
                 we1_ref, be1_ref, nh_ref, eh_ref, ei_ref, mij_ref,
                 *, rb, k, nrbf):
    d = jnp.transpose(eo_ref[0, 0:k, :])
    idxf = jnp.transpose(eo_ref[0, k:2 * k, :])
    validt = jnp.transpose(mi_ref[0])
    mij = (d < BIGV * 0.5).astype(jnp.float32) * validt
    mij_ref[0] = mij
    kio = jax.lax.broadcasted_iota(jnp.int32, (rb, k), 1)
    ei_ref[0] = jnp.where(validt > 0, idxf.astype(jnp.int32), kio)
    xc = jnp.transpose(x_ref[0])
    nh = (xc * wn_ref[0:1, :] + jnp.sin(xc) * wn_ref[1:2, :]
          + jnp.cos(xc) * wn_ref[2:3, :] + bn_ref[...])
    nh_ref[0] = nh * validt
    sigma = 5.0 / (nrbf - 1)
    centers = jax.lax.broadcasted_iota(
        jnp.int32, (1, nrbf), 1).astype(jnp.float32) * sigma
    rbf_parts, ff_parts = [], []
    for kk in range(k):
        dc = d[:, kk:kk + 1]
        rbf_parts.append(jnp.exp(-(((dc - centers) / sigma) ** 2)))
        ff_parts.append(dc * we1_ref[0:1, :]
                        + jnp.log(dc + 1e-3) * we1_ref[1:2, :])
    rbf = jnp.concatenate(rbf_parts, axis=0)
    em = jnp.dot(rbf, we0_ref[...], preferred_element_type=jnp.float32,
                 precision=jax.lax.Precision.HIGHEST)
    ff = jnp.concatenate(ff_parts, axis=0)
    eh = em + be0_ref[...] + ff + be1_ref[...]
    for kk in range(k):
        eh_ref[0, :, kk, :] = (eh[kk * rb:(kk + 1) * rb, :]
                               * mij[:, kk:kk + 1])


def kernel(X, C, W_n0, b_n0, W_e0, b_e0, W_e1, b_e1):
    B, N = C.shape
    K = KNN
    DE = W_e0.shape[1]
    DN = W_n0.shape[1]
    x = X[..., 0].reshape(B, 1, N)
    C3 = C.reshape(B, 1, N)
    RB, PB, NB, PC, RB2 = 256, 512, 512, 512, 256

    rank, mask_i = pl.pallas_call(
        functools.partial(_rank_kernel, rb=RB, n=N),
        grid=(B, N // RB),
        in_specs=[pl.BlockSpec((1, 1, N), lambda b, i: (b, 0, 0)),
                  pl.BlockSpec((1, 1, N), lambda b, i: (b, 0, 0))],
        out_specs=[pl.BlockSpec((1, 1, RB), lambda b, i: (b, 0, i)),
                   pl.BlockSpec((1, 1, RB), lambda b, i: (b, 0, i))],
        out_shape=[jax.ShapeDtypeStruct((B, 1, N), jnp.int32),
                   jax.ShapeDtypeStruct((B, 1, N), jnp.float32)],
    )(x, C3)

    sv, si = pl.pallas_call(
        functools.partial(_sortapply_kernel, pb=PB, n=N),
        grid=(B, N // PB),
        in_specs=[pl.BlockSpec((1, 1, N), lambda b, i: (b, 0, 0)),
                  pl.BlockSpec((1, 1, N), lambda b, i: (b, 0, 0)),
                  pl.BlockSpec((1, 1, N), lambda b, i: (b, 0, 0))],
        out_specs=[pl.BlockSpec((1, 1, PB), lambda b, i: (b, 0, i)),
                   pl.BlockSpec((1, 1, PB), lambda b, i: (b, 0, i))],
        out_shape=[jax.ShapeDtypeStruct((B, 1, N), jnp.float32),
                   jax.ShapeDtypeStruct((B, 1, N), jnp.int32)],
    )(rank, x, C3)

    svp = jnp.pad(sv, ((0, 0), (0, 0), (K, K)), constant_values=BIGV)
    sip = jnp.pad(si, ((0, 0), (0, 0), (K, K)))

    etab = pl.pallas_call(
        functools.partial(_knn_kernel, pb=PB, k=K),
        grid=(B, N // PB),
        in_specs=[pl.BlockSpec((1, 1, N + 2 * K), lambda b, i: (b, 0, 0)),
                  pl.BlockSpec((1, 1, N + 2 * K), lambda b, i: (b, 0, 0))],
        out_specs=pl.BlockSpec((1, 2 * K, PB), lambda b, i: (b, 0, i)),
        out_shape=jax.ShapeDtypeStruct((B, 2 * K, N), jnp.float32),
    )(svp, sip)

    eorig = pl.pallas_call(
        functools.partial(_gatherback_kernel, nb=NB, n=N, pc=PC, k=K),
        grid=(B, N // NB),
        in_specs=[pl.BlockSpec((1, 2 * K, N), lambda b, i: (b, 0, 0)),
                  pl.BlockSpec((1, 1, NB), lambda b, i: (b, 0, i))],
        out_specs=pl.BlockSpec((1, 2 * K, NB), lambda b, i: (b, 0, i)),
        out_shape=jax.ShapeDtypeStruct((B, 2 * K, N), jnp.float32),
    )(etab, rank)

    wspec = lambda shape: pl.BlockSpec(shape, lambda b, i: tuple(0 for _ in shape))
    node_h, edge_h, edge_idx, mask_ij = pl.pallas_call(
        functools.partial(_feat_kernel, rb=RB2, k=K, nrbf=NRBF),
        grid=(B, N // RB2),
        in_specs=[pl.BlockSpec((1, 2 * K, RB2), lambda b, i: (b, 0, i)),
                  pl.BlockSpec((1, 1, RB2), lambda b, i: (b, 0, i)),
                  pl.BlockSpec((1, 1, RB2), lambda b, i: (b, 0, i)),
                  wspec((3, DN)), wspec((1, DN)), wspec((NRBF, DE)),
                  wspec((1, DE)), wspec((2, DE)), wspec((1, DE))],
        out_specs=[pl.BlockSpec((1, RB2, DN), lambda b, i: (b, i, 0)),
                   pl.BlockSpec((1, RB2, K, DE), lambda b, i: (b, i, 0, 0)),
                   pl.BlockSpec((1, RB2, K), lambda b, i: (b, i, 0)),
                   pl.BlockSpec((1, RB2, K), lambda b, i: (b, i, 0))],
        out_shape=[jax.ShapeDtypeStruct((B, N, DN), jnp.float32),
                   jax.ShapeDtypeStruct((B, N, K, DE), jnp.float32),
                   jax.ShapeDtypeStruct((B, N, K), jnp.int32),
                   jax.ShapeDtypeStruct((B, N, K), jnp.float32)],
    )(eorig, x, mask_i, W_n0, b_n0.reshape(1, DN), W_e0,
      b_e0.reshape(1, DE), W_e1, b_e1.reshape(1, DE))

    return node_h, edge_h, edge_idx, mask_i.reshape(B, N), mask_ij


# speedup vs baseline: 21.2437x; 1.4617x over previous
import functools

import jax
import jax.numpy as jnp
from jax.experimental import pallas as pl

BIGV = 1e6
KNN = 32
NRBF = 16


def _rank_kernel(x_ref, c_ref, rank_ref, mi_ref, *, rb, n):
    i = pl.program_id(1)
    key = jnp.where(c_ref[0] > 0, x_ref[0], BIGV)
    ci = c_ref[0, 0:1, pl.ds(i * rb, rb)]
    xi = x_ref[0, 0:1, pl.ds(i * rb, rb)]
    ki = jnp.transpose(jnp.where(ci > 0, xi, BIGV))
    jg = jax.lax.broadcasted_iota(jnp.int32, (1, n), 1)
    ig = jax.lax.broadcasted_iota(jnp.int32, (rb, 1), 0) + i * rb
    less = (key < ki) | ((key == ki) & (jg < ig))
    rank = jnp.sum(less.astype(jnp.int32), axis=1, keepdims=True)
    rank_ref[0] = jnp.transpose(rank)
    mi_ref[0] = (ci > 0).astype(jnp.float32)


def _sortapply_kernel(rank_ref, x_ref, c_ref, sv_ref, si_ref, *, pb, n):
    p = pl.program_id(1)
    key = jnp.where(c_ref[0] > 0, x_ref[0], BIGV)
    rank = rank_ref[0]
    pg = jax.lax.broadcasted_iota(jnp.int32, (pb, 1), 0) + p * pb
    oh = rank == pg
    sv = jnp.sum(jnp.where(oh, key, 0.0), axis=1, keepdims=True)
    jg = jax.lax.broadcasted_iota(jnp.int32, (1, n), 1)
    si = jnp.sum(jnp.where(oh, jg, 0), axis=1, keepdims=True)
    sv_ref[0] = jnp.transpose(sv)
    si_ref[0] = jnp.transpose(si)


def _knn_kernel(svp_ref, sip_ref, et_ref, *, pb, k):
    p0 = pl.program_id(1) * pb
    win_v = svp_ref[0, 0:1, pl.ds(p0, pb + 2 * k)]
    win_i = sip_ref[0, 0:1, pl.ds(p0, pb + 2 * k)]
    rows_v = [win_v[:, w:w + pb] for w in range(2 * k + 1)]
    rows_i = [win_i[:, w:w + pb] for w in range(2 * k + 1)]
    pvals = rows_v[k]
    cv = jnp.concatenate(rows_v[:k] + rows_v[k + 1:], axis=0)
    cidx = jnp.concatenate(rows_i[:k] + rows_i[k + 1:], axis=0)
    dist = jnp.abs(cv - pvals)
    vals, idxs = [], []
    for _ in range(k):
        gmin = jnp.min(dist, axis=0, keepdims=True)
        tie = dist == gmin
        cho = jnp.min(jnp.where(tie, cidx, 2 ** 30), axis=0, keepdims=True)
        vals.append(gmin)
        idxs.append(cho)
        dist = jnp.where(tie & (cidx == cho), 3e6, dist)
    m = jnp.concatenate(
        vals + [v.astype(jnp.float32) for v in idxs], axis=0)
    et_ref[0] = jnp.transpose(m)


def _gatherback_kernel(et_ref, rank_ref, eo_ref, *, nb, n, pc, k):
    acc = jnp.zeros((nb, 2 * k), jnp.float32)
    rankc = jnp.transpose(rank_ref[0])
    for cp in range(n // pc):
        pg = jax.lax.broadcasted_iota(jnp.int32, (1, pc), 1) + cp * pc
        oh = (rankc == pg).astype(jnp.float32)
        e = et_ref[0, cp * pc:(cp + 1) * pc, :]
        acc = acc + jnp.dot(oh, e, preferred_element_type=jnp.float32,
                            precision=jax.lax.Precision.HIGHEST)
    eo_ref[0] = acc


def _feat_kernel(eo_ref, x_ref, mi_ref, wn_ref, bn_ref, we0_ref, be0_ref,
                 we1_ref, be1_ref, nh_ref, eh_ref, ei_ref, mij_ref,
                 *, rb, k, nrbf):
    de = we0_ref.shape[1]
    d = eo_ref[0, :, 0:k]
    idxf = eo_ref[0, :, k:2 * k]
    validt = jnp.transpose(mi_ref[0])
    mij = (d < BIGV * 0.5).astype(jnp.float32) * validt
    mij_ref[0] = mij
    kio = jax.lax.broadcasted_iota(jnp.int32, (rb, k), 1)
    ei_ref[0] = jnp.where(validt > 0, idxf.astype(jnp.int32), kio)
    xc = jnp.transpose(x_ref[0])
    nf = jnp.concatenate([xc, jnp.sin(xc), jnp.cos(xc)], axis=1)
    nh = jnp.dot(nf, wn_ref[...], preferred_element_type=jnp.float32,
                 precision=jax.lax.Precision.HIGHEST) + bn_ref[...]
    nh_ref[0] = nh * validt
    sigma = 5.0 / (nrbf - 1)
    centers = jax.lax.broadcasted_iota(
        jnp.int32, (1, 1, nrbf), 2).astype(jnp.float32) * sigma
    d1 = d.reshape(rb, k, 1)
    rbf = jnp.exp(-(((d1 - centers) / sigma) ** 2))
    em = jnp.dot(rbf.reshape(rb * k, nrbf), we0_ref[...],
                 preferred_element_type=jnp.float32)
    w10 = we1_ref[0:1, :].reshape(1, 1, de)
    w11 = we1_ref[1:2, :].reshape(1, 1, de)
    ff = d1 * w10 + jnp.log(d1 + 1e-3) * w11
    eh = (em.reshape(rb, k, de) + be0_ref[...].reshape(1, 1, de)
          + ff + be1_ref[...].reshape(1, 1, de))
    eh_ref[0] = eh * mij.reshape(rb, k, 1)


def kernel(X, C, W_n0, b_n0, W_e0, b_e0, W_e1, b_e1):
    B, N = C.shape
    K = KNN
    DE = W_e0.shape[1]
    DN = W_n0.shape[1]
    x = X[..., 0].reshape(B, 1, N)
    C3 = C.reshape(B, 1, N)
    RB, PB, NB, PC, RB2 = 256, 512, 512, 512, 256

    rank, mask_i = pl.pallas_call(
        functools.partial(_rank_kernel, rb=RB, n=N),
        grid=(B, N // RB),
        in_specs=[pl.BlockSpec((1, 1, N), lambda b, i: (b, 0, 0)),
                  pl.BlockSpec((1, 1, N), lambda b, i: (b, 0, 0))],
        out_specs=[pl.BlockSpec((1, 1, RB), lambda b, i: (b, 0, i)),
                   pl.BlockSpec((1, 1, RB), lambda b, i: (b, 0, i))],
        out_shape=[jax.ShapeDtypeStruct((B, 1, N), jnp.int32),
                   jax.ShapeDtypeStruct((B, 1, N), jnp.float32)],
    )(x, C3)

    sv, si = pl.pallas_call(
        functools.partial(_sortapply_kernel, pb=PB, n=N),
        grid=(B, N // PB),
        in_specs=[pl.BlockSpec((1, 1, N), lambda b, i: (b, 0, 0)),
                  pl.BlockSpec((1, 1, N), lambda b, i: (b, 0, 0)),
                  pl.BlockSpec((1, 1, N), lambda b, i: (b, 0, 0))],
        out_specs=[pl.BlockSpec((1, 1, PB), lambda b, i: (b, 0, i)),
                   pl.BlockSpec((1, 1, PB), lambda b, i: (b, 0, i))],
        out_shape=[jax.ShapeDtypeStruct((B, 1, N), jnp.float32),
                   jax.ShapeDtypeStruct((B, 1, N), jnp.int32)],
    )(rank, x, C3)

    svp = jnp.pad(sv, ((0, 0), (0, 0), (K, K)), constant_values=BIGV)
    sip = jnp.pad(si, ((0, 0), (0, 0), (K, K)))

    etab = pl.pallas_call(
        functools.partial(_knn_kernel, pb=PB, k=K),
        grid=(B, N // PB),
        in_specs=[pl.BlockSpec((1, 1, N + 2 * K), lambda b, i: (b, 0, 0)),
                  pl.BlockSpec((1, 1, N + 2 * K), lambda b, i: (b, 0, 0))],
        out_specs=pl.BlockSpec((1, PB, 2 * K), lambda b, i: (b, i, 0)),
        out_shape=jax.ShapeDtypeStruct((B, N, 2 * K), jnp.float32),
    )(svp, sip)

    eorig = pl.pallas_call(
        functools.partial(_gatherback_kernel, nb=NB, n=N, pc=PC, k=K),
        grid=(B, N // NB),
        in_specs=[pl.BlockSpec((1, N, 2 * K), lambda b, i: (b, 0, 0)),
                  pl.BlockSpec((1, 1, NB), lambda b, i: (b, 0, i))],
        out_specs=pl.BlockSpec((1, NB, 2 * K), lambda b, i: (b, i, 0)),
        out_shape=jax.ShapeDtypeStruct((B, N, 2 * K), jnp.float32),
    )(etab, rank)

    wspec = lambda shape: pl.BlockSpec(shape, lambda b, i: tuple(0 for _ in shape))
    node_h, edge_h, edge_idx, mask_ij = pl.pallas_call(
        functools.partial(_feat_kernel, rb=RB2, k=K, nrbf=NRBF),
        grid=(B, N // RB2),
        in_specs=[pl.BlockSpec((1, RB2, 2 * K), lambda b, i: (b, i, 0)),
                  pl.BlockSpec((1, 1, RB2), lambda b, i: (b, 0, i)),
                  pl.BlockSpec((1, 1, RB2), lambda b, i: (b, 0, i)),
                  wspec((3, DN)), wspec((1, DN)), wspec((NRBF, DE)),
                  wspec((1, DE)), wspec((2, DE)), wspec((1, DE))],
        out_specs=[pl.BlockSpec((1, RB2, DN), lambda b, i: (b, i, 0)),
                   pl.BlockSpec((1, RB2, K, DE), lambda b, i: (b, i, 0, 0)),
                   pl.BlockSpec((1, RB2, K), lambda b, i: (b, i, 0)),
                   pl.BlockSpec((1, RB2, K), lambda b, i: (b, i, 0))],
        out_shape=[jax.ShapeDtypeStruct((B, N, DN), jnp.float32),
                   jax.ShapeDtypeStruct((B, N, K, DE), jnp.float32),
                   jax.ShapeDtypeStruct((B, N, K), jnp.int32),
                   jax.ShapeDtypeStruct((B, N, K), jnp.float32)],
    )(eorig, x, mask_i, W_n0, b_n0.reshape(1, DN), W_e0,
      b_e0.reshape(1, DE), W_e1, b_e1.reshape(1, DE))

    return node_h, edge_h, edge_idx, mask_i.reshape(B, N), mask_ij


# speedup vs baseline: 28.6265x; 1.3475x over previous
import functools

import jax
import jax.numpy as jnp
from jax import lax
from jax.experimental import pallas as pl
from jax.experimental.pallas import tpu as pltpu
from jax.experimental.pallas import tpu_sc as plsc

BIGV = 1e6
KNN = 32
NRBF = 16


def _rank_kernel(x_ref, c_ref, rank_ref, mi_ref, *, rb, n):
    i = pl.program_id(1)
    key = jnp.where(c_ref[0] > 0, x_ref[0], BIGV)
    ci = c_ref[0, 0:1, pl.ds(i * rb, rb)]
    xi = x_ref[0, 0:1, pl.ds(i * rb, rb)]
    ki = jnp.transpose(jnp.where(ci > 0, xi, BIGV))
    jg = jax.lax.broadcasted_iota(jnp.int32, (1, n), 1)
    ig = jax.lax.broadcasted_iota(jnp.int32, (rb, 1), 0) + i * rb
    less = (key < ki) | ((key == ki) & (jg < ig))
    rank = jnp.sum(less.astype(jnp.int32), axis=1, keepdims=True)
    rank_ref[0] = jnp.transpose(rank)
    mi_ref[0] = (ci > 0).astype(jnp.float32)


def _sortapply_kernel(rank_ref, x_ref, c_ref, sv_ref, si_ref, *, pb, n):
    p = pl.program_id(1)
    key = jnp.where(c_ref[0] > 0, x_ref[0], BIGV)
    rank = rank_ref[0]
    pg = jax.lax.broadcasted_iota(jnp.int32, (pb, 1), 0) + p * pb
    oh = rank == pg
    sv = jnp.sum(jnp.where(oh, key, 0.0), axis=1, keepdims=True)
    jg = jax.lax.broadcasted_iota(jnp.int32, (1, n), 1)
    si = jnp.sum(jnp.where(oh, jg, 0), axis=1, keepdims=True)
    sv_ref[0] = jnp.transpose(sv)
    si_ref[0] = jnp.transpose(si)


def _knn_kernel(svp_ref, sip_ref, et_ref, *, pb, k):
    p0 = pl.program_id(1) * pb
    win_v = svp_ref[0, 0:1, pl.ds(p0, pb + 2 * k)]
    win_i = sip_ref[0, 0:1, pl.ds(p0, pb + 2 * k)]
    rows_v = [win_v[:, w:w + pb] for w in range(2 * k + 1)]
    rows_i = [win_i[:, w:w + pb] for w in range(2 * k + 1)]
    pvals = rows_v[k]
    cv = jnp.concatenate(rows_v[:k] + rows_v[k + 1:], axis=0)
    cidx = jnp.concatenate(rows_i[:k] + rows_i[k + 1:], axis=0)
    dist = jnp.abs(cv - pvals)
    vals, idxs = [], []
    for _ in range(k):
        gmin = jnp.min(dist, axis=0, keepdims=True)
        tie = dist == gmin
        cho = jnp.min(jnp.where(tie, cidx, 2 ** 30), axis=0, keepdims=True)
        vals.append(gmin)
        idxs.append(cho)
        dist = jnp.where(tie & (cidx == cho), 3e6, dist)
    m = jnp.concatenate(
        vals + [v.astype(jnp.float32) for v in idxs], axis=0)
    et_ref[0, :, 0:2 * k] = jnp.transpose(m)
    et_ref[0, :, 2 * k:] = jnp.zeros((pb, 2 * k), jnp.float32)


def _sc_gather_kernel(table_hbm, idx_hbm, out_hbm, idx_v, rows_v, sem,
                      *, b_per_w, ch, nc):
    wid = lax.axis_index("s") * nc + lax.axis_index("c")
    base = wid * b_per_w
    pltpu.sync_copy(idx_hbm.at[pl.ds(base, b_per_w)], idx_v)
    for j in range(b_per_w // ch):
        pltpu.async_copy(
            table_hbm.at[idx_v.at[pl.ds(j * ch, ch)]],
            rows_v.at[pl.ds(j * ch, ch)], sem).wait()
    pltpu.sync_copy(rows_v, out_hbm.at[pl.ds(base, b_per_w)])


def _feat_kernel(eo_ref, x_ref, mi_ref, wn_ref, bn_ref, we0_ref, be0_ref,
                 we1_ref, be1_ref, nh_ref, eh_ref, ei_ref, mij_ref,
                 *, rb, k, nrbf):
    de = we0_ref.shape[1]
    d = eo_ref[0, :, 0:k]
    idxf = eo_ref[0, :, k:2 * k]
    validt = jnp.transpose(mi_ref[0])
    mij = (d < BIGV * 0.5).astype(jnp.float32) * validt
    mij_ref[0] = mij
    kio = jax.lax.broadcasted_iota(jnp.int32, (rb, k), 1)
    ei_ref[0] = jnp.where(validt > 0, idxf.astype(jnp.int32), kio)
    xc = jnp.transpose(x_ref[0])
    nf = jnp.concatenate([xc, jnp.sin(xc), jnp.cos(xc)], axis=1)
    nh = jnp.dot(nf, wn_ref[...], preferred_element_type=jnp.float32,
                 precision=jax.lax.Precision.HIGHEST) + bn_ref[...]
    nh_ref[0] = nh * validt
    sigma = 5.0 / (nrbf - 1)
    centers = jax.lax.broadcasted_iota(
        jnp.int32, (1, 1, nrbf), 2).astype(jnp.float32) * sigma
    d1 = d.reshape(rb, k, 1)
    rbf = jnp.exp(-(((d1 - centers) / sigma) ** 2))
    em = jnp.dot(rbf.reshape(rb * k, nrbf), we0_ref[...],
                 preferred_element_type=jnp.float32)
    w10 = we1_ref[0:1, :].reshape(1, 1, de)
    w11 = we1_ref[1:2, :].reshape(1, 1, de)
    ff = d1 * w10 + jnp.log(d1 + 1e-3) * w11
    eh = (em.reshape(rb, k, de) + be0_ref[...].reshape(1, 1, de)
          + ff + be1_ref[...].reshape(1, 1, de))
    eh_ref[0] = eh * mij.reshape(rb, k, 1)


def kernel(X, C, W_n0, b_n0, W_e0, b_e0, W_e1, b_e1):
    B, N = C.shape
    K = KNN
    DE = W_e0.shape[1]
    DN = W_n0.shape[1]
    x = X[..., 0].reshape(B, 1, N)
    C3 = C.reshape(B, 1, N)
    RB, PB, NB, PC, RB2 = 256, 512, 512, 512, 256

    rank, mask_i = pl.pallas_call(
        functools.partial(_rank_kernel, rb=RB, n=N),
        grid=(B, N // RB),
        in_specs=[pl.BlockSpec((1, 1, N), lambda b, i: (b, 0, 0)),
                  pl.BlockSpec((1, 1, N), lambda b, i: (b, 0, 0))],
        out_specs=[pl.BlockSpec((1, 1, RB), lambda b, i: (b, 0, i)),
                   pl.BlockSpec((1, 1, RB), lambda b, i: (b, 0, i))],
        out_shape=[jax.ShapeDtypeStruct((B, 1, N), jnp.int32),
                   jax.ShapeDtypeStruct((B, 1, N), jnp.float32)],
    )(x, C3)

    sv, si = pl.pallas_call(
        functools.partial(_sortapply_kernel, pb=PB, n=N),
        grid=(B, N // PB),
        in_specs=[pl.BlockSpec((1, 1, N), lambda b, i: (b, 0, 0)),
                  pl.BlockSpec((1, 1, N), lambda b, i: (b, 0, 0)),
                  pl.BlockSpec((1, 1, N), lambda b, i: (b, 0, 0))],
        out_specs=[pl.BlockSpec((1, 1, PB), lambda b, i: (b, 0, i)),
                   pl.BlockSpec((1, 1, PB), lambda b, i: (b, 0, i))],
        out_shape=[jax.ShapeDtypeStruct((B, 1, N), jnp.float32),
                   jax.ShapeDtypeStruct((B, 1, N), jnp.int32)],
    )(rank, x, C3)

    svp = jnp.pad(sv, ((0, 0), (0, 0), (K, K)), constant_values=BIGV)
    sip = jnp.pad(si, ((0, 0), (0, 0), (K, K)))

    etab = pl.pallas_call(
        functools.partial(_knn_kernel, pb=PB, k=K),
        grid=(B, N // PB),
        in_specs=[pl.BlockSpec((1, 1, N + 2 * K), lambda b, i: (b, 0, 0)),
                  pl.BlockSpec((1, 1, N + 2 * K), lambda b, i: (b, 0, 0))],
        out_specs=pl.BlockSpec((1, PB, 4 * K), lambda b, i: (b, i, 0)),
        out_shape=jax.ShapeDtypeStruct((B, N, 4 * K), jnp.float32),
    )(svp, sip)

    info = plsc.get_sparse_core_info()
    nw = info.num_cores * info.num_subcores
    b_per_w = (B * N) // nw
    mesh = plsc.VectorSubcoreMesh(core_axis_name="c", subcore_axis_name="s")
    table = etab.reshape(B * N, 4 * K)
    idxg = (rank[:, 0, :]
            + jnp.arange(B, dtype=jnp.int32)[:, None] * N).reshape(B * N)
    sc_gather = pl.kernel(
        functools.partial(_sc_gather_kernel, b_per_w=b_per_w, ch=128,
                          nc=info.num_cores),
        out_type=jax.ShapeDtypeStruct((B * N, 4 * K), jnp.float32),
        mesh=mesh,
        scratch_types=[pltpu.VMEM((b_per_w,), jnp.int32),
                       pltpu.VMEM((b_per_w, 4 * K), jnp.float32),
                       pltpu.SemaphoreType.DMA],
    )
    eorig = sc_gather(table, idxg).reshape(B, N, 4 * K)

    wspec = lambda shape: pl.BlockSpec(shape, lambda b, i: tuple(0 for _ in shape))
    node_h, edge_h, edge_idx, mask_ij = pl.pallas_call(
        functools.partial(_feat_kernel, rb=RB2, k=K, nrbf=NRBF),
        grid=(B, N // RB2),
        in_specs=[pl.BlockSpec((1, RB2, 4 * K), lambda b, i: (b, i, 0)),
                  pl.BlockSpec((1, 1, RB2), lambda b, i: (b, 0, i)),
                  pl.BlockSpec((1, 1, RB2), lambda b, i: (b, 0, i)),
                  wspec((3, DN)), wspec((1, DN)), wspec((NRBF, DE)),
                  wspec((1, DE)), wspec((2, DE)), wspec((1, DE))],
        out_specs=[pl.BlockSpec((1, RB2, DN), lambda b, i: (b, i, 0)),
                   pl.BlockSpec((1, RB2, K, DE), lambda b, i: (b, i, 0, 0)),
                   pl.BlockSpec((1, RB2, K), lambda b, i: (b, i, 0)),
                   pl.BlockSpec((1, RB2, K), lambda b, i: (b, i, 0))],
        out_shape=[jax.ShapeDtypeStruct((B, N, DN), jnp.float32),
                   jax.ShapeDtypeStruct((B, N, K, DE), jnp.float32),
                   jax.ShapeDtypeStruct((B, N, K), jnp.int32),
                   jax.ShapeDtypeStruct((B, N, K), jnp.float32)],
    )(eorig, x, mask_i, W_n0, b_n0.reshape(1, DN), W_e0,
      b_e0.reshape(1, DE), W_e1, b_e1.reshape(1, DE))

    return node_h, edge_h, edge_idx, mask_i.reshape(B, N), mask_ij
